# baseline (speedup 1.0000x reference)
import math

import jax
import jax.numpy as jnp
from jax.experimental import pallas as pl

_B, _K, _D = 4096, 50, 128
_BB = 32
_TWO_GAMMA = 2.0 * 0.57721566490153286060
_C = math.exp(-_TWO_GAMMA)
_EPS = 1e-13
_INV_N = 1.0 / (_B * _K)


def _duck_block(ent_ref, rel_ref, ecnt_ref, ncnt_ref, out_ref):
    i = pl.program_id(0)

    e = ent_ref[...]
    el_exp = jnp.exp(e[:, :_D])
    er_exp = jnp.exp(-e[:, _D:])

    val = rel_ref[...]
    val4 = val.reshape(_BB, _K, 2, _D)
    p = jnp.exp(val4[:, :, 0, :])
    q = jnp.exp(-val4[:, :, 1, :])

    a = el_exp[:, None, :] + p
    ar = er_exp[:, None, :] + q
    pq = p * q
    aar = a * ar
    u = 1.0 / (pq * aar)
    t_i = _C * (pq * u)
    t_r = _C * (aar * u)

    elem = jnp.log(jnp.log1p(t_r) + _EPS) - jnp.log(jnp.log1p(t_i) + _EPS)
    row = jnp.sum(elem, axis=-1)

    mask = (ncnt_ref[...] >= 1) & (ecnt_ref[...] >= 1)
    part = jnp.sum(jnp.where(mask, row, 0.0)) * _INV_N

    @pl.when(i == 0)
    def _():
        out_ref[...] = jnp.zeros_like(out_ref)

    out_ref[...] += part.reshape(1, 1)


def kernel(entity_box, rel_box, entity_rel_counts, neighbor_rel_counts):
    ent = entity_box.reshape(_B, 2 * _D)
    rel = rel_box.reshape(_B, 2 * _K, _D)
    ecnt = entity_rel_counts.reshape(_B, 1)

    out = pl.pallas_call(
        _duck_block,
        grid=(_B // _BB,),
        in_specs=[
            pl.BlockSpec((_BB, 2 * _D), lambda i: (i, 0)),
            pl.BlockSpec((_BB, 2 * _K, _D), lambda i: (i, 0, 0)),
            pl.BlockSpec((_BB, 1), lambda i: (i, 0)),
            pl.BlockSpec((_BB, _K), lambda i: (i, 0)),
        ],
        out_specs=pl.BlockSpec((1, 1), lambda i: (0, 0)),
        out_shape=jax.ShapeDtypeStruct((1, 1), jnp.float32),
    )(ent, rel, ecnt, neighbor_rel_counts)
    return out[0, 0]


# speedup vs baseline: 1.4656x; 1.4656x over previous
import math

import jax
import jax.numpy as jnp
from jax.experimental import pallas as pl

_B, _K, _D = 4096, 50, 128
_NC = 8
_CB = _B // _NC
_BB = 128
_TWO_GAMMA = 2.0 * 0.57721566490153286060
_C = math.exp(-_TWO_GAMMA)
_EPS = 1e-13
_INV_N = 1.0 / (_B * _K)


def _duck_block(ent_ref, rel_ref, ecnt_ref, ncnt_ref, out_ref):
    i = pl.program_id(0)

    e = ent_ref[...]
    el_exp = jnp.exp(e[:, :_D])
    er_exp = jnp.exp(-e[:, _D:])

    rel = rel_ref[...]
    p = jnp.exp(rel[:, :, :_D])
    q = jnp.exp(-rel[:, :, _D:])

    a = el_exp[:, None, :] + p
    ar = er_exp[:, None, :] + q
    pq = p * q
    aar = a * ar
    u = 1.0 / (pq * aar)
    t_i = _C * (pq * u)
    t_r = _C * (aar * u)

    elem = jnp.log(jnp.log1p(t_r) + _EPS) - jnp.log(jnp.log1p(t_i) + _EPS)
    row = jnp.sum(elem, axis=-1)

    mask = (ncnt_ref[...] >= 1) & (ecnt_ref[...] >= 1)
    part = jnp.sum(jnp.where(mask, row, 0.0)) * _INV_N

    @pl.when(i == 0)
    def _():
        out_ref[...] = jnp.zeros_like(out_ref)

    out_ref[...] += part.reshape(1, 1)


def _chunk_sum(ent, rel, ecnt, ncnt):
    return pl.pallas_call(
        _duck_block,
        grid=(_CB // _BB,),
        in_specs=[
            pl.BlockSpec((_BB, 2 * _D), lambda i: (i, 0)),
            pl.BlockSpec((_BB, _K, 2 * _D), lambda i: (i, 0, 0)),
            pl.BlockSpec((_BB, 1), lambda i: (i, 0)),
            pl.BlockSpec((_BB, _K), lambda i: (i, 0)),
        ],
        out_specs=pl.BlockSpec((1, 1), lambda i: (0, 0)),
        out_shape=jax.ShapeDtypeStruct((1, 1), jnp.float32),
    )(ent, rel, ecnt, ncnt)


def kernel(entity_box, rel_box, entity_rel_counts, neighbor_rel_counts):
    ent = entity_box.reshape(_B, 2 * _D)
    ecnt = entity_rel_counts.reshape(_B, 1)

    parts = []
    for c in range(_NC):
        lo = c * _CB
        rel_c = jax.lax.slice_in_dim(rel_box, lo, lo + _CB, axis=0)
        rel_c = rel_c.reshape(_CB, _K, 2 * _D)
        parts.append(
            _chunk_sum(
                jax.lax.slice_in_dim(ent, lo, lo + _CB, axis=0),
                rel_c,
                jax.lax.slice_in_dim(ecnt, lo, lo + _CB, axis=0),
                jax.lax.slice_in_dim(neighbor_rel_counts, lo, lo + _CB, axis=0),
            )
        )
    total = parts[0]
    for ppart in parts[1:]:
        total = total + ppart
    return total[0, 0]


# speedup vs baseline: 2.5241x; 1.7222x over previous
import math

import jax
import jax.numpy as jnp
from jax.experimental import pallas as pl

_B, _K, _D = 4096, 50, 128
_NC = 8
_CB = _B // _NC
_BB = 128
_TWO_GAMMA = 2.0 * 0.57721566490153286060
_C = math.exp(-_TWO_GAMMA)
_EPS = 1e-13
_INV_N = 1.0 / (_B * _K)


def _duck_block(ent_ref, rel_ref, ecnt_ref, ncnt_ref, out_ref):
    i = pl.program_id(0)

    el_exp = jnp.exp(ent_ref[:, 0, :])
    er_exp = jnp.exp(-ent_ref[:, 1, :])

    rel = rel_ref[...]
    p = jnp.exp(rel[:, :, :_D])
    q = jnp.exp(-rel[:, :, _D:])

    a = el_exp[:, None, :] + p
    ar = er_exp[:, None, :] + q
    pq = p * q
    aar = a * ar
    u = 1.0 / (pq * aar)
    t_i = _C * (pq * u)
    t_r = _C * (aar * u)

    elem = jnp.log(jnp.log1p(t_r) + _EPS) - jnp.log(jnp.log1p(t_i) + _EPS)
    row = jnp.sum(elem, axis=-1)

    mask = (ncnt_ref[...] >= 1) & (ecnt_ref[...] >= 1)
    part = jnp.sum(jnp.where(mask, row, 0.0)) * _INV_N

    @pl.when(i == 0)
    def _():
        out_ref[...] = jnp.zeros_like(out_ref)

    out_ref[...] += part.reshape(1, 1)


def kernel(entity_box, rel_box, entity_rel_counts, neighbor_rel_counts):
    rel = rel_box.reshape(_B, _K, 2 * _D)
    ecnt = entity_rel_counts.reshape(_B, 1)

    out = pl.pallas_call(
        _duck_block,
        grid=(_B // _BB,),
        in_specs=[
            pl.BlockSpec((_BB, 2, _D), lambda i: (i, 0, 0)),
            pl.BlockSpec((_BB, _K, 2 * _D), lambda i: (i, 0, 0)),
            pl.BlockSpec((_BB, 1), lambda i: (i, 0)),
            pl.BlockSpec((_BB, _K), lambda i: (i, 0)),
        ],
        out_specs=pl.BlockSpec((1, 1), lambda i: (0, 0)),
        out_shape=jax.ShapeDtypeStruct((1, 1), jnp.float32),
    )(entity_box, rel, ecnt, neighbor_rel_counts)
    return out[0, 0]


# speedup vs baseline: 4.0985x; 1.6238x over previous
import math

import jax
import jax.numpy as jnp
from jax.experimental import pallas as pl
from jax.experimental.pallas import tpu as pltpu

_B, _K, _D = 4096, 50, 128
_BB = 64
_TWO_GAMMA = 2.0 * 0.57721566490153286060
_C = math.exp(-_TWO_GAMMA)
_EPS = 1e-13
_INV_N = 1.0 / (_B * _K)


def _duck_block(ent_ref, rel_ref, ecnt_ref, ncnt_ref, out_ref,
                buf_ref, sem_l, sem_r):
    i = pl.program_id(0)

    copy_l = pltpu.make_async_copy(
        rel_ref.at[:, :, 0, :], buf_ref.at[:, :, pl.ds(0, _D)], sem_l)
    copy_r = pltpu.make_async_copy(
        rel_ref.at[:, :, 1, :], buf_ref.at[:, :, pl.ds(_D, _D)], sem_r)
    copy_l.start()
    copy_r.start()

    el_exp = jnp.exp(ent_ref[:, 0, :])
    er_exp = jnp.exp(-ent_ref[:, 1, :])

    copy_l.wait()
    copy_r.wait()

    rel = buf_ref[...]
    p = jnp.exp(rel[:, :, :_D])
    q = jnp.exp(-rel[:, :, _D:])

    a = el_exp[:, None, :] + p
    ar = er_exp[:, None, :] + q
    pq = p * q
    aar = a * ar
    u = 1.0 / (pq * aar)
    t_i = _C * (pq * u)
    t_r = _C * (aar * u)

    elem = jnp.log(jnp.log1p(t_r) + _EPS) - jnp.log(jnp.log1p(t_i) + _EPS)
    row = jnp.sum(elem, axis=-1)

    mask = (ncnt_ref[...] >= 1) & (ecnt_ref[...] >= 1)
    part = jnp.sum(jnp.where(mask, row, 0.0)) * _INV_N

    @pl.when(i == 0)
    def _():
        out_ref[...] = jnp.zeros_like(out_ref)

    out_ref[...] += part.reshape(1, 1)


def kernel(entity_box, rel_box, entity_rel_counts, neighbor_rel_counts):
    ecnt = entity_rel_counts.reshape(_B, 1)

    out = pl.pallas_call(
        _duck_block,
        grid=(_B // _BB,),
        in_specs=[
            pl.BlockSpec((_BB, 2, _D), lambda i: (i, 0, 0)),
            pl.BlockSpec((_BB, _K, 2, _D), lambda i: (i, 0, 0, 0)),
            pl.BlockSpec((_BB, 1), lambda i: (i, 0)),
            pl.BlockSpec((_BB, _K), lambda i: (i, 0)),
        ],
        out_specs=pl.BlockSpec((1, 1), lambda i: (0, 0)),
        out_shape=jax.ShapeDtypeStruct((1, 1), jnp.float32),
        scratch_shapes=[
            pltpu.VMEM((_BB, _K, 2 * _D), jnp.float32),
            pltpu.SemaphoreType.DMA,
            pltpu.SemaphoreType.DMA,
        ],
    )(entity_box, rel_box, ecnt, neighbor_rel_counts)
    return out[0, 0]


# speedup vs baseline: 4.0999x; 1.0003x over previous
import math

import jax
import jax.numpy as jnp
from jax.experimental import pallas as pl
from jax.experimental.pallas import tpu as pltpu

_B, _K, _D = 4096, 50, 128
_BB = 64
_TWO_GAMMA = 2.0 * 0.57721566490153286060
_C = math.exp(-_TWO_GAMMA)
_EPS = 1e-13
_INV_N = 1.0 / (_B * _K)


def _duck_block(ent_ref, rel_ref, ecnt_ref, ncnt_ref, out_ref,
                buf_ref, sem_l, sem_r):
    i = pl.program_id(0)
    n = pl.num_programs(0) - 1
    slot = jax.lax.rem(i, 2)
    prev = jax.lax.rem(i + 1, 2)

    @pl.when(i < n)
    def _():
        pltpu.make_async_copy(
            rel_ref.at[:, :, 0, :],
            buf_ref.at[slot, :, :, pl.ds(0, _D)], sem_l.at[slot]).start()
        pltpu.make_async_copy(
            rel_ref.at[:, :, 1, :],
            buf_ref.at[slot, :, :, pl.ds(_D, _D)], sem_r.at[slot]).start()

    @pl.when(i == 0)
    def _():
        out_ref[...] = jnp.zeros_like(out_ref)

    @pl.when(i > 0)
    def _():
        pltpu.make_async_copy(
            rel_ref.at[:, :, 0, :],
            buf_ref.at[prev, :, :, pl.ds(0, _D)], sem_l.at[prev]).wait()
        pltpu.make_async_copy(
            rel_ref.at[:, :, 1, :],
            buf_ref.at[prev, :, :, pl.ds(_D, _D)], sem_r.at[prev]).wait()

        el_exp = jnp.exp(ent_ref[:, 0, :])
        er_exp = jnp.exp(-ent_ref[:, 1, :])

        rel = buf_ref[prev]
        p = jnp.exp(rel[:, :, :_D])
        q = jnp.exp(-rel[:, :, _D:])

        a = el_exp[:, None, :] + p
        ar = er_exp[:, None, :] + q
        pq = p * q
        aar = a * ar
        u = 1.0 / (pq * aar)
        t_i = _C * (pq * u)
        t_r = _C * (aar * u)

        elem = jnp.log(jnp.log1p(t_r) + _EPS) - jnp.log(jnp.log1p(t_i) + _EPS)
        row = jnp.sum(elem, axis=-1)

        mask = (ncnt_ref[...] >= 1) & (ecnt_ref[...] >= 1)
        part = jnp.sum(jnp.where(mask, row, 0.0)) * _INV_N
        out_ref[...] += part.reshape(1, 1)


def kernel(entity_box, rel_box, entity_rel_counts, neighbor_rel_counts):
    ecnt = entity_rel_counts.reshape(_B, 1)
    n = _B // _BB

    out = pl.pallas_call(
        _duck_block,
        grid=(n + 1,),
        in_specs=[
            pl.BlockSpec((_BB, 2, _D),
                         lambda i: (jnp.maximum(i - 1, 0), 0, 0)),
            pl.BlockSpec((_BB, _K, 2, _D),
                         lambda i: (jnp.minimum(i, n - 1), 0, 0, 0)),
            pl.BlockSpec((_BB, 1), lambda i: (jnp.maximum(i - 1, 0), 0)),
            pl.BlockSpec((_BB, _K), lambda i: (jnp.maximum(i - 1, 0), 0)),
        ],
        out_specs=pl.BlockSpec((1, 1), lambda i: (0, 0)),
        out_shape=jax.ShapeDtypeStruct((1, 1), jnp.float32),
        scratch_shapes=[
            pltpu.VMEM((2, _BB, _K, 2 * _D), jnp.float32),
            pltpu.SemaphoreType.DMA((2,)),
            pltpu.SemaphoreType.DMA((2,)),
        ],
    )(entity_box, rel_box, ecnt, neighbor_rel_counts)
    return out[0, 0]
